# Initial kernel scaffold; baseline (speedup 1.0000x reference)
import jax
import jax.numpy as jnp
from jax.experimental import pallas as pl


def kernel(X, weights):
    raise NotImplementedError("write your pallas kernel here")


# speedup vs baseline: 1.3098x; 1.3098x over previous
import functools

import jax
import jax.numpy as jnp
from jax import lax
from jax.experimental import pallas as pl
from jax.experimental.pallas import tpu as pltpu
from jax.experimental.pallas import tpu_sc as plsc

_NUM_CORES = 2
_NUM_SUBCORES = 16
_NW = _NUM_CORES * _NUM_SUBCORES

_CH = 128


@functools.lru_cache(maxsize=None)
def _make_gather(total, d):
    bpw = total // _NW
    nch = bpw // _CH
    mesh = plsc.VectorSubcoreMesh(core_axis_name="c", subcore_axis_name="s")

    @functools.partial(
        pl.kernel,
        mesh=mesh,
        compiler_params=pltpu.CompilerParams(use_tc_tiling_on_sc=False),
        out_type=jax.ShapeDtypeStruct((_NW, nch, _CH, d), jnp.float32),
        scratch_types=[
            pltpu.VMEM((nch, _CH), jnp.int32),
            pltpu.VMEM((2, _CH, d), jnp.float32),
            pltpu.SemaphoreType.DMA,
        ],
    )
    def gather(table_hbm, idx_hbm, out_hbm, idx_v, rows_v, gsem):
        wid = lax.axis_index("s") * _NUM_CORES + lax.axis_index("c")
        pltpu.sync_copy(idx_hbm.at[wid], idx_v)

        def body(j, carry):
            pltpu.async_copy(table_hbm.at[idx_v.at[j]], rows_v.at[0], gsem).wait()
            pltpu.sync_copy(rows_v.at[0], out_hbm.at[wid, j])
            return carry

        lax.fori_loop(0, nch, body, 0)

    return gather


def kernel(X, weights):
    b, h = X.shape
    d = weights.shape[1]
    total = b * h
    idx = jnp.reshape(X.astype(jnp.int32), (_NW, total // (_NW * _CH), _CH))
    out = _make_gather(total, d)(weights, idx)
    return jnp.reshape(out, (b, h, d))


# speedup vs baseline: 1.4962x; 1.1423x over previous
import functools

import jax
import jax.numpy as jnp
from jax import lax
from jax.experimental import pallas as pl
from jax.experimental.pallas import tpu as pltpu
from jax.experimental.pallas import tpu_sc as plsc

_NUM_CORES = 2
_NUM_SUBCORES = 16
_NW = _NUM_CORES * _NUM_SUBCORES

_CH = 128
_K = 10
_MEGA = _K * _CH


@functools.lru_cache(maxsize=None)
def _make_gather(total, d):
    bpw = total // _NW
    nch = bpw // _CH
    nmega = bpw // _MEGA
    mesh = plsc.VectorSubcoreMesh(core_axis_name="c", subcore_axis_name="s")

    @functools.partial(
        pl.kernel,
        mesh=mesh,
        compiler_params=pltpu.CompilerParams(use_tc_tiling_on_sc=False),
        out_type=jax.ShapeDtypeStruct((_NW, nmega, _MEGA, d), jnp.float32),
        scratch_types=[
            pltpu.VMEM((nch, _CH), jnp.int32),
            pltpu.VMEM((2, _MEGA, d), jnp.float32),
            pltpu.SemaphoreType.DMA,
            pltpu.SemaphoreType.DMA,
        ],
    )
    def gather(table_hbm, idx_hbm, out_hbm, idx_v, rows_v, gsem, osem):
        wid = lax.axis_index("s") * _NUM_CORES + lax.axis_index("c")
        pltpu.sync_copy(idx_hbm.at[wid], idx_v)

        def fire(m, slot):
            for b in range(_K):
                pltpu.async_copy(
                    table_hbm.at[idx_v.at[m * _K + b]],
                    rows_v.at[slot, pl.ds(b * _CH, _CH)],
                    gsem,
                )

        fire(0, 0)

        @pl.loop(0, nmega, step=2)
        def _(m0):
            for s in range(2):
                m = m0 + s
                pltpu.make_async_copy(out_hbm.at[wid, 0], rows_v.at[s], gsem).wait()

                @pl.when(m >= 1)
                def _():
                    pltpu.make_async_copy(
                        rows_v.at[1 - s], out_hbm.at[wid, 0], osem
                    ).wait()

                @pl.when(m + 1 < nmega)
                def _():
                    fire(m + 1, 1 - s)

                pltpu.async_copy(rows_v.at[s], out_hbm.at[wid, m], osem)

        pltpu.make_async_copy(
            rows_v.at[(nmega - 1) % 2], out_hbm.at[wid, 0], osem
        ).wait()

    return gather


def kernel(X, weights):
    b, h = X.shape
    d = weights.shape[1]
    total = b * h
    idx = jnp.reshape(X.astype(jnp.int32), (_NW, total // (_NW * _CH), _CH))
    out = _make_gather(total, d)(weights, idx)
    return jnp.reshape(out, (b, h, d))
